# baseline (speedup 1.0000x reference)
import functools

import jax
import jax.numpy as jnp
from jax import lax
from jax.experimental import pallas as pl
from jax.experimental.pallas import tpu as pltpu
from jax.experimental.pallas import tpu_sc as plsc

S, H, FF, E, TOPK = 2048, 768, 768, 8, 2
P = S * TOPK
BT = 256
NB = P // BT + E
C = NB * BT
RB = 128

NCORES, NSUB = 2, 16
NW = NCORES * NSUB
GPW = C // NW
CPW = S // NW


def _route_dispatch_body(ri_ref, rw_ref, logits_ref, pos_ref, w_ref, be_ref,
                         used_ref):
    ri = ri_ref[...]
    rw = rw_ref[...]
    logits = lax.dot_general(
        ri, rw, (((1,), (1,)), ((), ())), preferred_element_type=jnp.float32
    )
    logits_ref[...] = logits

    colid = lax.broadcasted_iota(jnp.int32, (S, E), 1)
    m1 = jnp.max(logits, axis=1, keepdims=True)
    idx1 = jnp.min(jnp.where(logits == m1, colid, E), axis=1, keepdims=True)
    l2 = jnp.where(colid == idx1, -jnp.inf, logits)
    m2 = jnp.max(l2, axis=1, keepdims=True)
    idx2 = jnp.min(jnp.where(l2 == m2, colid, E), axis=1, keepdims=True)
    e2 = jnp.exp(m2 - m1)
    denom = 1.0 + e2
    w_ref[...] = jnp.concatenate([1.0 / denom, e2 / denom], axis=1)

    one_hot1 = jnp.where(colid == idx1, 1.0, 0.0)
    one_hot2 = jnp.where(colid == idx2, 1.0, 0.0)
    x12 = jnp.concatenate([one_hot1, one_hot2], axis=1).astype(jnp.bfloat16)

    ri_b = lax.broadcasted_iota(jnp.int32, (RB, RB), 0)
    ci_b = lax.broadcasted_iota(jnp.int32, (RB, RB), 1)
    ls = jnp.where(ci_b < ri_b, 1.0, 0.0).astype(jnp.bfloat16)
    excl_parts = []
    offs = jnp.zeros((1, 2 * E), jnp.float32)
    for blk in range(S // RB):
        sub = x12[blk * RB:(blk + 1) * RB, :]
        part = lax.dot_general(
            ls, sub, (((1,), (0,)), ((), ())),
            preferred_element_type=jnp.float32)
        excl_parts.append(part + offs)
        offs = offs + jnp.sum(sub.astype(jnp.float32), axis=0, keepdims=True)
    excl = jnp.concatenate(excl_parts, axis=0)
    excl1 = excl[:, :E]
    excl2 = excl[:, E:]
    total1 = jnp.sum(one_hot1, axis=0, keepdims=True)
    total2 = jnp.sum(one_hot2, axis=0, keepdims=True)
    counts = total1 + total2

    nblk = jnp.floor((counts + (BT - 1)) * (1.0 / BT))
    er = lax.broadcasted_iota(jnp.int32, (E, E), 0)
    ec = lax.broadcasted_iota(jnp.int32, (E, E), 1)
    m8 = jnp.where(er < ec, 1.0, 0.0)
    blkoff = lax.dot_general(
        nblk, m8, (((1,), (0,)), ((), ())),
        precision=lax.Precision.HIGHEST,
        preferred_element_type=jnp.float32)
    padoff = blkoff * float(BT)
    total_blocks = jnp.sum(nblk, axis=1, keepdims=True)

    def sel(tab, idx):
        t = jnp.broadcast_to(tab, (S, E))
        return jnp.sum(jnp.where(colid == idx, t, 0.0), axis=1, keepdims=True)

    rank0 = jnp.sum(jnp.where(colid == idx1, excl1, 0.0), axis=1,
                    keepdims=True)
    rank1 = jnp.sum(jnp.where(colid == idx2, excl2, 0.0), axis=1,
                    keepdims=True)
    pos0 = sel(padoff, idx1) + rank0
    pos1 = sel(padoff, idx2) + sel(total1, idx2) + rank1
    pos_ref[...] = jnp.concatenate([pos0, pos1], axis=1).astype(jnp.int32)

    br = lax.broadcasted_iota(jnp.int32, (NB, E), 0).astype(jnp.float32)
    bo = jnp.broadcast_to(blkoff, (NB, E))
    be = jnp.sum(jnp.where(bo <= br, 1.0, 0.0), axis=1, keepdims=True) - 1.0
    be_ref[...] = be.astype(jnp.int32)
    brow = lax.broadcasted_iota(jnp.int32, (NB, 1), 0).astype(jnp.float32)
    used_ref[...] = jnp.where(
        brow < jnp.broadcast_to(total_blocks, (NB, 1)), 1, 0
    ).astype(jnp.int32)


def _inverse_perm_body(pos_ref, w_ref, src_ref, wsort_ref):
    b = pl.program_id(0)
    pos = pos_ref[...]
    rid = lax.broadcasted_iota(jnp.int32, (P, BT), 1) + b * BT
    pm = jnp.where(jnp.broadcast_to(pos, (P, BT)) == rid, 1.0, 0.0)
    tval = (lax.broadcasted_iota(jnp.int32, (P, 1), 0) % S).astype(jnp.float32)
    tw = jnp.concatenate([tval, w_ref[...]], axis=1)
    res = lax.dot_general(
        pm, tw, (((0,), (0,)), ((), ())),
        precision=lax.Precision.HIGHEST,
        preferred_element_type=jnp.float32)
    src_ref[...] = res[:, 0:1].astype(jnp.int32)
    wsort_ref[...] = res[:, 1:2]


def _grouped_expert_body(be_sref, used_sref, x_ref, w_ref, wg_ref, wu_ref,
                         wd_ref, y_ref):
    b = pl.program_id(0)

    @pl.when(used_sref[b] == 1)
    def _():
        x = x_ref[...].astype(jnp.bfloat16)
        dims = (((1,), (0,)), ((), ()))
        g = lax.dot_general(x, wg_ref[0], dims,
                            preferred_element_type=jnp.float32)
        u = lax.dot_general(x, wu_ref[0], dims,
                            preferred_element_type=jnp.float32)
        a = (jnp.maximum(g, 0.0) * u).astype(jnp.bfloat16)
        d = lax.dot_general(a, wd_ref[0], dims,
                            preferred_element_type=jnp.float32)
        y_ref[...] = d * w_ref[...]


@functools.lru_cache(maxsize=None)
def _sc_gather_kernel():
    mesh = plsc.VectorSubcoreMesh(core_axis_name="c", subcore_axis_name="s")

    @functools.partial(
        pl.kernel,
        mesh=mesh,
        out_type=jax.ShapeDtypeStruct((C, H), jnp.float32),
        scratch_types=[
            pltpu.VMEM((GPW // 2,), jnp.int32),
            pltpu.VMEM((GPW // 2, H), jnp.float32),
            pltpu.SemaphoreType.DMA,
        ],
    )
    def _sc_gather(hidden_hbm, src_hbm, out_hbm, idx_v, rows_v, sem):
        wid = lax.axis_index("s") * NCORES + lax.axis_index("c")
        base = wid * GPW
        for half in range(2):
            off = base + half * (GPW // 2)
            pltpu.sync_copy(src_hbm.at[pl.ds(off, GPW // 2)], idx_v)
            pltpu.async_copy(hidden_hbm.at[idx_v], rows_v, sem).wait()
            pltpu.sync_copy(rows_v, out_hbm.at[pl.ds(off, GPW // 2)])

    return _sc_gather


@functools.lru_cache(maxsize=None)
def _sc_combine_kernel():
    mesh = plsc.VectorSubcoreMesh(core_axis_name="c", subcore_axis_name="s")

    @functools.partial(
        pl.kernel,
        mesh=mesh,
        out_type=jax.ShapeDtypeStruct((S, H), jnp.float32),
        scratch_types=[
            pltpu.VMEM((CPW,), jnp.int32),
            pltpu.VMEM((CPW,), jnp.int32),
            pltpu.VMEM((CPW, H), jnp.float32),
            pltpu.VMEM((CPW, H), jnp.float32),
            pltpu.SemaphoreType.DMA,
        ],
    )
    def _sc_combine(y_hbm, pos_hbm, out_hbm, idx0, idx1, buf0, buf1, sem):
        wid = lax.axis_index("s") * NCORES + lax.axis_index("c")
        base = wid * CPW
        pltpu.sync_copy(pos_hbm.at[pl.ds(base, CPW)], idx0)
        pltpu.sync_copy(pos_hbm.at[pl.ds(S + base, CPW)], idx1)
        pltpu.async_copy(y_hbm.at[idx0], buf0, sem).wait()
        pltpu.async_copy(y_hbm.at[idx1], buf1, sem).wait()

        def row_body(r, carry):
            for c in range(H // 16):
                sl = pl.ds(c * 16, 16)
                buf0[r, sl] = buf0[r, sl] + buf1[r, sl]
            return carry

        lax.fori_loop(0, CPW, row_body, 0)
        pltpu.sync_copy(buf0, out_hbm.at[pl.ds(base, CPW)])

    return _sc_combine


@jax.jit
def kernel(router_input, hidden_states, router_w, w_gate, w_up, w_down):
    logits, pos, w, be, used = pl.pallas_call(
        _route_dispatch_body,
        out_shape=(
            jax.ShapeDtypeStruct((S, E), jnp.float32),
            jax.ShapeDtypeStruct((S, TOPK), jnp.int32),
            jax.ShapeDtypeStruct((S, TOPK), jnp.float32),
            jax.ShapeDtypeStruct((NB, 1), jnp.int32),
            jax.ShapeDtypeStruct((NB, 1), jnp.int32),
        ),
    )(router_input, router_w)

    pos_flat = jnp.transpose(pos).reshape(P)
    pos_col = pos_flat.reshape(P, 1)
    w_col = jnp.transpose(w).reshape(P, 1)

    src_col, wsort = pl.pallas_call(
        _inverse_perm_body,
        grid=(NB,),
        in_specs=[
            pl.BlockSpec((P, 1), lambda b: (0, 0)),
            pl.BlockSpec((P, 1), lambda b: (0, 0)),
        ],
        out_specs=(
            pl.BlockSpec((BT, 1), lambda b: (b, 0)),
            pl.BlockSpec((BT, 1), lambda b: (b, 0)),
        ),
        out_shape=(
            jax.ShapeDtypeStruct((C, 1), jnp.int32),
            jax.ShapeDtypeStruct((C, 1), jnp.float32),
        ),
    )(pos_col, w_col)

    xs = _sc_gather_kernel()(hidden_states, src_col.reshape(C))

    wg = w_gate.astype(jnp.bfloat16)
    wu = w_up.astype(jnp.bfloat16)
    wd = w_down.astype(jnp.bfloat16)

    y = pl.pallas_call(
        _grouped_expert_body,
        grid_spec=pltpu.PrefetchScalarGridSpec(
            num_scalar_prefetch=2,
            grid=(NB,),
            in_specs=[
                pl.BlockSpec((BT, H), lambda b, be_r, u_r: (b, 0)),
                pl.BlockSpec((BT, 1), lambda b, be_r, u_r: (b, 0)),
                pl.BlockSpec((1, H, FF), lambda b, be_r, u_r: (be_r[b], 0, 0)),
                pl.BlockSpec((1, H, FF), lambda b, be_r, u_r: (be_r[b], 0, 0)),
                pl.BlockSpec((1, FF, H), lambda b, be_r, u_r: (be_r[b], 0, 0)),
            ],
            out_specs=pl.BlockSpec((BT, H), lambda b, be_r, u_r: (b, 0)),
        ),
        out_shape=jax.ShapeDtypeStruct((C, H), jnp.float32),
    )(be.reshape(NB), used.reshape(NB), xs, wsort, wg, wu, wd)

    out = _sc_combine_kernel()(y, pos_flat)
    return (out, logits)


# speedup vs baseline: 2.6947x; 2.6947x over previous
import functools

import jax
import jax.numpy as jnp
from jax import lax
from jax.experimental import pallas as pl
from jax.experimental.pallas import tpu as pltpu
from jax.experimental.pallas import tpu_sc as plsc

S, H, FF, E, TOPK = 2048, 768, 768, 8, 2
P = S * TOPK
BT = 256
NB = P // BT + E
C = NB * BT
RB = 128

NCORES, NSUB = 2, 16
NW = NCORES * NSUB
CPW = S // NW


def _route_dispatch_body(ri_ref, rw_ref, x_ref, logits_ref, pos_ref,
                         xsc0_ref, xsc1_ref, be_ref, used_ref):
    ri = ri_ref[...]
    rw = rw_ref[...]
    logits = lax.dot_general(
        ri, rw, (((1,), (1,)), ((), ())), preferred_element_type=jnp.float32
    )
    logits_ref[...] = logits

    colid = lax.broadcasted_iota(jnp.int32, (S, E), 1)
    m1 = jnp.max(logits, axis=1, keepdims=True)
    idx1 = jnp.min(jnp.where(logits == m1, colid, E), axis=1, keepdims=True)
    l2 = jnp.where(colid == idx1, -jnp.inf, logits)
    m2 = jnp.max(l2, axis=1, keepdims=True)
    idx2 = jnp.min(jnp.where(l2 == m2, colid, E), axis=1, keepdims=True)
    e2 = jnp.exp(m2 - m1)
    denom = 1.0 + e2
    w0 = 1.0 / denom
    w1 = e2 / denom

    x = x_ref[...]
    xsc0_ref[...] = x * jnp.sqrt(w0)
    xsc1_ref[...] = x * jnp.sqrt(w1)

    one_hot1 = jnp.where(colid == idx1, 1.0, 0.0)
    one_hot2 = jnp.where(colid == idx2, 1.0, 0.0)
    x12 = jnp.concatenate([one_hot1, one_hot2], axis=1).astype(jnp.bfloat16)

    ri_b = lax.broadcasted_iota(jnp.int32, (RB, RB), 0)
    ci_b = lax.broadcasted_iota(jnp.int32, (RB, RB), 1)
    ls = jnp.where(ci_b < ri_b, 1.0, 0.0).astype(jnp.bfloat16)
    excl_parts = []
    offs = jnp.zeros((1, 2 * E), jnp.float32)
    for blk in range(S // RB):
        sub = x12[blk * RB:(blk + 1) * RB, :]
        part = lax.dot_general(
            ls, sub, (((1,), (0,)), ((), ())),
            preferred_element_type=jnp.float32)
        excl_parts.append(part + offs)
        offs = offs + jnp.sum(sub.astype(jnp.float32), axis=0, keepdims=True)
    excl = jnp.concatenate(excl_parts, axis=0)
    excl1 = excl[:, :E]
    excl2 = excl[:, E:]
    total1 = jnp.sum(one_hot1, axis=0, keepdims=True)
    total2 = jnp.sum(one_hot2, axis=0, keepdims=True)
    counts = total1 + total2

    nblk = jnp.floor((counts + (BT - 1)) * (1.0 / BT))
    er = lax.broadcasted_iota(jnp.int32, (E, E), 0)
    ec = lax.broadcasted_iota(jnp.int32, (E, E), 1)
    m8 = jnp.where(er < ec, 1.0, 0.0)
    blkoff = lax.dot_general(
        nblk, m8, (((1,), (0,)), ((), ())),
        precision=lax.Precision.HIGHEST,
        preferred_element_type=jnp.float32)
    padoff = blkoff * float(BT)
    total_blocks = jnp.sum(nblk, axis=1, keepdims=True)

    def sel(tab, idx):
        t = jnp.broadcast_to(tab, (S, E))
        return jnp.sum(jnp.where(colid == idx, t, 0.0), axis=1, keepdims=True)

    rank0 = jnp.sum(jnp.where(colid == idx1, excl1, 0.0), axis=1,
                    keepdims=True)
    rank1 = jnp.sum(jnp.where(colid == idx2, excl2, 0.0), axis=1,
                    keepdims=True)
    pos0 = sel(padoff, idx1) + rank0
    pos1 = sel(padoff, idx2) + sel(total1, idx2) + rank1
    pos_ref[...] = jnp.concatenate([pos0, pos1], axis=1).astype(jnp.int32)

    br = lax.broadcasted_iota(jnp.int32, (NB, E), 0).astype(jnp.float32)
    bo = jnp.broadcast_to(blkoff, (NB, E))
    be = jnp.sum(jnp.where(bo <= br, 1.0, 0.0), axis=1, keepdims=True) - 1.0
    be_ref[...] = be.astype(jnp.int32)
    brow = lax.broadcasted_iota(jnp.int32, (NB, 1), 0).astype(jnp.float32)
    used_ref[...] = jnp.where(
        brow < jnp.broadcast_to(total_blocks, (NB, 1)), 1, 0
    ).astype(jnp.int32)


def _grouped_expert_body(be_sref, used_sref, x_ref, wg_ref, wu_ref,
                         wd_ref, y_ref):
    b = pl.program_id(0)

    @pl.when(used_sref[b] == 1)
    def _():
        x = x_ref[...].astype(jnp.bfloat16)
        dims = (((1,), (0,)), ((), ()))
        g = lax.dot_general(x, wg_ref[0], dims,
                            preferred_element_type=jnp.float32)
        u = lax.dot_general(x, wu_ref[0], dims,
                            preferred_element_type=jnp.float32)
        a = (jnp.maximum(g, 0.0) * u).astype(jnp.bfloat16)
        y_ref[...] = lax.dot_general(a, wd_ref[0], dims,
                                     preferred_element_type=jnp.float32)


@functools.lru_cache(maxsize=None)
def _sc_scatter_kernel():
    mesh = plsc.VectorSubcoreMesh(core_axis_name="c", subcore_axis_name="s")

    @functools.partial(
        pl.kernel,
        mesh=mesh,
        out_type=jax.ShapeDtypeStruct((C, H), jnp.float32),
        scratch_types=[
            pltpu.VMEM((CPW,), jnp.int32),
            pltpu.VMEM((CPW,), jnp.int32),
            pltpu.VMEM((CPW, H), jnp.float32),
            pltpu.VMEM((CPW, H), jnp.float32),
            pltpu.SemaphoreType.DMA,
        ],
    )
    def _sc_scatter(xsc0_hbm, xsc1_hbm, pos_hbm, out_hbm, idx0, idx1,
                    rows0, rows1, sem):
        wid = lax.axis_index("s") * NCORES + lax.axis_index("c")
        base = wid * CPW
        pltpu.sync_copy(pos_hbm.at[pl.ds(base, CPW)], idx0)
        pltpu.sync_copy(pos_hbm.at[pl.ds(S + base, CPW)], idx1)
        pltpu.sync_copy(xsc0_hbm.at[pl.ds(base, CPW)], rows0)
        pltpu.sync_copy(xsc1_hbm.at[pl.ds(base, CPW)], rows1)
        pltpu.async_copy(rows0, out_hbm.at[idx0], sem).wait()
        pltpu.async_copy(rows1, out_hbm.at[idx1], sem).wait()

    return _sc_scatter


@functools.lru_cache(maxsize=None)
def _sc_combine_kernel():
    mesh = plsc.VectorSubcoreMesh(core_axis_name="c", subcore_axis_name="s")

    @functools.partial(
        pl.kernel,
        mesh=mesh,
        out_type=jax.ShapeDtypeStruct((S, H), jnp.float32),
        scratch_types=[
            pltpu.VMEM((CPW,), jnp.int32),
            pltpu.VMEM((CPW,), jnp.int32),
            pltpu.VMEM((CPW, H), jnp.float32),
            pltpu.VMEM((CPW, H), jnp.float32),
            pltpu.SemaphoreType.DMA,
        ],
    )
    def _sc_combine(y_hbm, pos_hbm, out_hbm, idx0, idx1, buf0, buf1, sem):
        wid = lax.axis_index("s") * NCORES + lax.axis_index("c")
        base = wid * CPW
        pltpu.sync_copy(pos_hbm.at[pl.ds(base, CPW)], idx0)
        pltpu.sync_copy(pos_hbm.at[pl.ds(S + base, CPW)], idx1)
        pltpu.async_copy(y_hbm.at[idx0], buf0, sem).wait()
        pltpu.async_copy(y_hbm.at[idx1], buf1, sem).wait()

        def row_body(r, carry):
            for c in range(H // 16):
                sl = pl.ds(c * 16, 16)
                buf0[r, sl] = buf0[r, sl] + buf1[r, sl]
            return carry

        lax.fori_loop(0, CPW, row_body, 0)
        pltpu.sync_copy(buf0, out_hbm.at[pl.ds(base, CPW)])

    return _sc_combine


@jax.jit
def kernel(router_input, hidden_states, router_w, w_gate, w_up, w_down):
    logits, pos, xsc0, xsc1, be, used = pl.pallas_call(
        _route_dispatch_body,
        out_shape=(
            jax.ShapeDtypeStruct((S, E), jnp.float32),
            jax.ShapeDtypeStruct((S, TOPK), jnp.int32),
            jax.ShapeDtypeStruct((S, H), jnp.float32),
            jax.ShapeDtypeStruct((S, H), jnp.float32),
            jax.ShapeDtypeStruct((NB, 1), jnp.int32),
            jax.ShapeDtypeStruct((NB, 1), jnp.int32),
        ),
    )(router_input, router_w, hidden_states)

    pos_flat = jnp.transpose(pos).reshape(P)

    xs = _sc_scatter_kernel()(xsc0, xsc1, pos_flat)

    wg = w_gate.astype(jnp.bfloat16)
    wu = w_up.astype(jnp.bfloat16)
    wd = w_down.astype(jnp.bfloat16)

    y = pl.pallas_call(
        _grouped_expert_body,
        grid_spec=pltpu.PrefetchScalarGridSpec(
            num_scalar_prefetch=2,
            grid=(NB,),
            in_specs=[
                pl.BlockSpec((BT, H), lambda b, be_r, u_r: (b, 0)),
                pl.BlockSpec((1, H, FF), lambda b, be_r, u_r: (be_r[b], 0, 0)),
                pl.BlockSpec((1, H, FF), lambda b, be_r, u_r: (be_r[b], 0, 0)),
                pl.BlockSpec((1, FF, H), lambda b, be_r, u_r: (be_r[b], 0, 0)),
            ],
            out_specs=pl.BlockSpec((BT, H), lambda b, be_r, u_r: (b, 0)),
        ),
        out_shape=jax.ShapeDtypeStruct((C, H), jnp.float32),
    )(be.reshape(NB), used.reshape(NB), xs, wg, wu, wd)

    out = _sc_combine_kernel()(y, pos_flat)
    return (out, logits)


# speedup vs baseline: 3.1027x; 1.1514x over previous
import functools

import jax
import jax.numpy as jnp
from jax import lax
from jax.experimental import pallas as pl
from jax.experimental.pallas import tpu as pltpu
from jax.experimental.pallas import tpu_sc as plsc

S, H, FF, E, TOPK = 2048, 768, 768, 8, 2
P = S * TOPK
BT = 256
NB = P // BT + E
C = NB * BT
RB = 128

NCORES, NSUB = 2, 16
NW = NCORES * NSUB
CPW = S // NW


def _route_dispatch_body(ri_ref, rw_ref, x_ref, logits_ref, pos_ref,
                         xsc0_ref, xsc1_ref, be_ref, used_ref):
    ri = ri_ref[...]
    rw = rw_ref[...]
    logits = lax.dot_general(
        ri, rw, (((1,), (1,)), ((), ())), preferred_element_type=jnp.float32
    )
    logits_ref[...] = logits

    colid = lax.broadcasted_iota(jnp.int32, (S, E), 1)
    m1 = jnp.max(logits, axis=1, keepdims=True)
    idx1 = jnp.min(jnp.where(logits == m1, colid, E), axis=1, keepdims=True)
    l2 = jnp.where(colid == idx1, -jnp.inf, logits)
    m2 = jnp.max(l2, axis=1, keepdims=True)
    idx2 = jnp.min(jnp.where(l2 == m2, colid, E), axis=1, keepdims=True)
    e2 = jnp.exp(m2 - m1)
    denom = 1.0 + e2
    w0 = 1.0 / denom
    w1 = e2 / denom

    x = x_ref[...]
    xsc0_ref[...] = x * jnp.sqrt(w0)
    xsc1_ref[...] = x * jnp.sqrt(w1)

    one_hot1 = jnp.where(colid == idx1, 1.0, 0.0)
    one_hot2 = jnp.where(colid == idx2, 1.0, 0.0)
    x12 = jnp.concatenate([one_hot1, one_hot2], axis=1).astype(jnp.bfloat16)

    ri_b = lax.broadcasted_iota(jnp.int32, (RB, RB), 0)
    ci_b = lax.broadcasted_iota(jnp.int32, (RB, RB), 1)
    ls = jnp.where(ci_b < ri_b, 1.0, 0.0).astype(jnp.bfloat16)
    excl_parts = []
    offs = jnp.zeros((1, 2 * E), jnp.float32)
    for blk in range(S // RB):
        sub = x12[blk * RB:(blk + 1) * RB, :]
        part = lax.dot_general(
            ls, sub, (((1,), (0,)), ((), ())),
            preferred_element_type=jnp.float32)
        excl_parts.append(part + offs)
        offs = offs + jnp.sum(sub.astype(jnp.float32), axis=0, keepdims=True)
    excl = jnp.concatenate(excl_parts, axis=0)
    excl1 = excl[:, :E]
    excl2 = excl[:, E:]
    total1 = jnp.sum(one_hot1, axis=0, keepdims=True)
    total2 = jnp.sum(one_hot2, axis=0, keepdims=True)
    counts = total1 + total2

    nblk = jnp.floor((counts + (BT - 1)) * (1.0 / BT))
    er = lax.broadcasted_iota(jnp.int32, (E, E), 0)
    ec = lax.broadcasted_iota(jnp.int32, (E, E), 1)
    m8 = jnp.where(er < ec, 1.0, 0.0)
    blkoff = lax.dot_general(
        nblk, m8, (((1,), (0,)), ((), ())),
        precision=lax.Precision.HIGHEST,
        preferred_element_type=jnp.float32)
    padoff = blkoff * float(BT)
    total_blocks = jnp.sum(nblk, axis=1, keepdims=True)

    def sel(tab, idx):
        t = jnp.broadcast_to(tab, (S, E))
        return jnp.sum(jnp.where(colid == idx, t, 0.0), axis=1, keepdims=True)

    rank0 = jnp.sum(jnp.where(colid == idx1, excl1, 0.0), axis=1,
                    keepdims=True)
    rank1 = jnp.sum(jnp.where(colid == idx2, excl2, 0.0), axis=1,
                    keepdims=True)
    pos0 = sel(padoff, idx1) + rank0
    pos1 = sel(padoff, idx2) + sel(total1, idx2) + rank1
    pos_ref[...] = jnp.concatenate([pos0, pos1], axis=1).astype(jnp.int32)

    br = lax.broadcasted_iota(jnp.int32, (NB, E), 0).astype(jnp.float32)
    bo = jnp.broadcast_to(blkoff, (NB, E))
    be = jnp.sum(jnp.where(bo <= br, 1.0, 0.0), axis=1, keepdims=True) - 1.0
    be_ref[...] = be.astype(jnp.int32)
    brow = lax.broadcasted_iota(jnp.int32, (NB, 1), 0).astype(jnp.float32)
    used_ref[...] = jnp.where(
        brow < jnp.broadcast_to(total_blocks, (NB, 1)), 1, 0
    ).astype(jnp.int32)


def _grouped_expert_body(be_sref, used_sref, x_ref, wg_ref, wu_ref,
                         wd_ref, y_ref):
    b = pl.program_id(0)

    @pl.when(used_sref[b] == 1)
    def _():
        x = x_ref[...]
        dims = (((1,), (0,)), ((), ()))
        g = lax.dot_general(x, wg_ref[0], dims,
                            preferred_element_type=jnp.float32)
        u = lax.dot_general(x, wu_ref[0], dims,
                            preferred_element_type=jnp.float32)
        a = jnp.maximum(g, 0.0) * u
        y_ref[...] = lax.dot_general(a, wd_ref[0], dims,
                                     preferred_element_type=jnp.float32)


@functools.lru_cache(maxsize=None)
def _sc_scatter_kernel():
    mesh = plsc.VectorSubcoreMesh(core_axis_name="c", subcore_axis_name="s")

    @functools.partial(
        pl.kernel,
        mesh=mesh,
        out_type=jax.ShapeDtypeStruct((C, H), jnp.float32),
        scratch_types=[
            pltpu.VMEM((CPW,), jnp.int32),
            pltpu.VMEM((CPW,), jnp.int32),
            pltpu.VMEM((CPW, H), jnp.float32),
            pltpu.VMEM((CPW, H), jnp.float32),
            pltpu.SemaphoreType.DMA,
        ],
    )
    def _sc_scatter(xsc0_hbm, xsc1_hbm, pos_hbm, out_hbm, idx0, idx1,
                    rows0, rows1, sem):
        wid = lax.axis_index("s") * NCORES + lax.axis_index("c")
        base = wid * CPW
        pltpu.sync_copy(pos_hbm.at[pl.ds(base, CPW)], idx0)
        pltpu.sync_copy(pos_hbm.at[pl.ds(S + base, CPW)], idx1)
        pltpu.sync_copy(xsc0_hbm.at[pl.ds(base, CPW)], rows0)
        pltpu.sync_copy(xsc1_hbm.at[pl.ds(base, CPW)], rows1)
        pltpu.async_copy(rows0, out_hbm.at[idx0], sem).wait()
        pltpu.async_copy(rows1, out_hbm.at[idx1], sem).wait()

    return _sc_scatter


@functools.lru_cache(maxsize=None)
def _sc_combine_kernel():
    mesh = plsc.VectorSubcoreMesh(core_axis_name="c", subcore_axis_name="s")

    @functools.partial(
        pl.kernel,
        mesh=mesh,
        out_type=jax.ShapeDtypeStruct((S, H), jnp.float32),
        scratch_types=[
            pltpu.VMEM((CPW,), jnp.int32),
            pltpu.VMEM((CPW,), jnp.int32),
            pltpu.VMEM((CPW, H), jnp.float32),
            pltpu.VMEM((CPW, H), jnp.float32),
            pltpu.SemaphoreType.DMA,
        ],
    )
    def _sc_combine(y_hbm, pos_hbm, out_hbm, idx0, idx1, buf0, buf1, sem):
        wid = lax.axis_index("s") * NCORES + lax.axis_index("c")
        base = wid * CPW
        pltpu.sync_copy(pos_hbm.at[pl.ds(base, CPW)], idx0)
        pltpu.sync_copy(pos_hbm.at[pl.ds(S + base, CPW)], idx1)
        pltpu.async_copy(y_hbm.at[idx0], buf0, sem).wait()
        pltpu.async_copy(y_hbm.at[idx1], buf1, sem).wait()

        def row_body(r, carry):
            for c in range(H // 16):
                sl = pl.ds(c * 16, 16)
                buf0[r, sl] = buf0[r, sl] + buf1[r, sl]
            return carry

        lax.fori_loop(0, CPW, row_body, 0)
        pltpu.sync_copy(buf0, out_hbm.at[pl.ds(base, CPW)])

    return _sc_combine


@jax.jit
def kernel(router_input, hidden_states, router_w, w_gate, w_up, w_down):
    logits, pos, xsc0, xsc1, be, used = pl.pallas_call(
        _route_dispatch_body,
        out_shape=(
            jax.ShapeDtypeStruct((S, E), jnp.float32),
            jax.ShapeDtypeStruct((S, TOPK), jnp.int32),
            jax.ShapeDtypeStruct((S, H), jnp.float32),
            jax.ShapeDtypeStruct((S, H), jnp.float32),
            jax.ShapeDtypeStruct((NB, 1), jnp.int32),
            jax.ShapeDtypeStruct((NB, 1), jnp.int32),
        ),
    )(router_input, router_w, hidden_states)

    pos_flat = jnp.transpose(pos).reshape(P)

    xs = _sc_scatter_kernel()(xsc0, xsc1, pos_flat)

    y = pl.pallas_call(
        _grouped_expert_body,
        grid_spec=pltpu.PrefetchScalarGridSpec(
            num_scalar_prefetch=2,
            grid=(NB,),
            in_specs=[
                pl.BlockSpec((BT, H), lambda b, be_r, u_r: (b, 0)),
                pl.BlockSpec((1, H, FF), lambda b, be_r, u_r: (be_r[b], 0, 0)),
                pl.BlockSpec((1, H, FF), lambda b, be_r, u_r: (be_r[b], 0, 0)),
                pl.BlockSpec((1, FF, H), lambda b, be_r, u_r: (be_r[b], 0, 0)),
            ],
            out_specs=pl.BlockSpec((BT, H), lambda b, be_r, u_r: (b, 0)),
        ),
        out_shape=jax.ShapeDtypeStruct((C, H), jnp.float32),
    )(be.reshape(NB), used.reshape(NB), xs, w_gate, w_up, w_down)

    out = _sc_combine_kernel()(y, pos_flat)
    return (out, logits)


# speedup vs baseline: 3.1363x; 1.0108x over previous
import functools

import jax
import jax.numpy as jnp
from jax import lax
from jax.experimental import pallas as pl
from jax.experimental.pallas import tpu as pltpu
from jax.experimental.pallas import tpu_sc as plsc

S, H, FF, E, TOPK = 2048, 768, 768, 8, 2
P = S * TOPK
BT = 256
NB = P // BT + E
C = NB * BT
RB = 128

NCORES, NSUB = 2, 16
NW = NCORES * NSUB
CPW = S // NW


def _route_dispatch_body(ri_ref, rw_ref, x_ref, logits_ref, pos_ref,
                         xsc0_ref, xsc1_ref, be_ref, used_ref):
    ri = ri_ref[...]
    rw = rw_ref[...]
    logits = lax.dot_general(
        ri, rw, (((1,), (1,)), ((), ())), preferred_element_type=jnp.float32
    )
    logits_ref[...] = logits

    colid = lax.broadcasted_iota(jnp.int32, (S, E), 1)
    m1 = jnp.max(logits, axis=1, keepdims=True)
    idx1 = jnp.min(jnp.where(logits == m1, colid, E), axis=1, keepdims=True)
    l2 = jnp.where(colid == idx1, -jnp.inf, logits)
    m2 = jnp.max(l2, axis=1, keepdims=True)
    idx2 = jnp.min(jnp.where(l2 == m2, colid, E), axis=1, keepdims=True)
    e2 = jnp.exp(m2 - m1)
    denom = 1.0 + e2
    w0 = 1.0 / denom
    w1 = e2 / denom

    x = x_ref[...]
    xsc0_ref[...] = x * jnp.sqrt(w0)
    xsc1_ref[...] = x * jnp.sqrt(w1)

    one_hot1 = jnp.where(colid == idx1, 1.0, 0.0)
    one_hot2 = jnp.where(colid == idx2, 1.0, 0.0)
    x12 = jnp.concatenate([one_hot1, one_hot2], axis=1).astype(jnp.bfloat16)

    ri_b = lax.broadcasted_iota(jnp.int32, (RB, RB), 0)
    ci_b = lax.broadcasted_iota(jnp.int32, (RB, RB), 1)
    ls = jnp.where(ci_b < ri_b, 1.0, 0.0).astype(jnp.bfloat16)
    excl_parts = []
    offs = jnp.zeros((1, 2 * E), jnp.float32)
    for blk in range(S // RB):
        sub = x12[blk * RB:(blk + 1) * RB, :]
        part = lax.dot_general(
            ls, sub, (((1,), (0,)), ((), ())),
            preferred_element_type=jnp.float32)
        excl_parts.append(part + offs)
        offs = offs + jnp.sum(sub.astype(jnp.float32), axis=0, keepdims=True)
    excl = jnp.concatenate(excl_parts, axis=0)
    excl1 = excl[:, :E]
    excl2 = excl[:, E:]
    total1 = jnp.sum(one_hot1, axis=0, keepdims=True)
    total2 = jnp.sum(one_hot2, axis=0, keepdims=True)
    counts = total1 + total2

    nblk = jnp.floor((counts + (BT - 1)) * (1.0 / BT))
    er = lax.broadcasted_iota(jnp.int32, (E, E), 0)
    ec = lax.broadcasted_iota(jnp.int32, (E, E), 1)
    m8 = jnp.where(er < ec, 1.0, 0.0)
    blkoff = lax.dot_general(
        nblk, m8, (((1,), (0,)), ((), ())),
        precision=lax.Precision.HIGHEST,
        preferred_element_type=jnp.float32)
    padoff = blkoff * float(BT)
    total_blocks = jnp.sum(nblk, axis=1, keepdims=True)

    def sel(tab, idx):
        t = jnp.broadcast_to(tab, (S, E))
        return jnp.sum(jnp.where(colid == idx, t, 0.0), axis=1, keepdims=True)

    rank0 = jnp.sum(jnp.where(colid == idx1, excl1, 0.0), axis=1,
                    keepdims=True)
    rank1 = jnp.sum(jnp.where(colid == idx2, excl2, 0.0), axis=1,
                    keepdims=True)
    pos0 = sel(padoff, idx1) + rank0
    pos1 = sel(padoff, idx2) + sel(total1, idx2) + rank1
    pos_ref[...] = jnp.concatenate([pos0, pos1], axis=1).astype(jnp.int32)

    br = lax.broadcasted_iota(jnp.int32, (NB, E), 0).astype(jnp.float32)
    bo = jnp.broadcast_to(blkoff, (NB, E))
    be = jnp.sum(jnp.where(bo <= br, 1.0, 0.0), axis=1, keepdims=True) - 1.0
    be_ref[...] = be.astype(jnp.int32)
    brow = lax.broadcasted_iota(jnp.int32, (NB, 1), 0).astype(jnp.float32)
    used_ref[...] = jnp.where(
        brow < jnp.broadcast_to(total_blocks, (NB, 1)), 1, 0
    ).astype(jnp.int32)


def _grouped_expert_body(be_sref, used_sref, x_ref, wg_ref, wu_ref,
                         wd_ref, y_ref):
    b = pl.program_id(0)

    @pl.when(used_sref[b] == 1)
    def _():
        x = x_ref[...]
        dims = (((1,), (0,)), ((), ()))
        g = lax.dot_general(x, wg_ref[0], dims,
                            preferred_element_type=jnp.float32)
        u = lax.dot_general(x, wu_ref[0], dims,
                            preferred_element_type=jnp.float32)
        a = jnp.maximum(g, 0.0) * u
        y_ref[...] = lax.dot_general(a, wd_ref[0], dims,
                                     preferred_element_type=jnp.float32)


@functools.lru_cache(maxsize=None)
def _sc_scatter_kernel():
    mesh = plsc.VectorSubcoreMesh(core_axis_name="c", subcore_axis_name="s")

    @functools.partial(
        pl.kernel,
        mesh=mesh,
        out_type=jax.ShapeDtypeStruct((C, H), jnp.float32),
        scratch_types=[
            pltpu.VMEM((CPW,), jnp.int32),
            pltpu.VMEM((CPW,), jnp.int32),
            pltpu.VMEM((CPW, H), jnp.float32),
            pltpu.VMEM((CPW, H), jnp.float32),
            pltpu.SemaphoreType.DMA,
            pltpu.SemaphoreType.DMA,
            pltpu.SemaphoreType.DMA,
        ],
    )
    def _sc_scatter(xsc0_hbm, xsc1_hbm, pos_hbm, out_hbm, idx0, idx1,
                    rows0, rows1, semr0, semr1, semw):
        wid = lax.axis_index("s") * NCORES + lax.axis_index("c")
        base = wid * CPW
        pltpu.sync_copy(pos_hbm.at[pl.ds(base, CPW)], idx0)
        pltpu.sync_copy(pos_hbm.at[pl.ds(S + base, CPW)], idx1)
        r0 = pltpu.async_copy(xsc0_hbm.at[pl.ds(base, CPW)], rows0, semr0)
        r1 = pltpu.async_copy(xsc1_hbm.at[pl.ds(base, CPW)], rows1, semr1)
        r0.wait()
        w0 = pltpu.async_copy(rows0, out_hbm.at[idx0], semw)
        r1.wait()
        w1 = pltpu.async_copy(rows1, out_hbm.at[idx1], semw)
        w0.wait()
        w1.wait()

    return _sc_scatter


@functools.lru_cache(maxsize=None)
def _sc_combine_kernel():
    mesh = plsc.VectorSubcoreMesh(core_axis_name="c", subcore_axis_name="s")

    @functools.partial(
        pl.kernel,
        mesh=mesh,
        out_type=jax.ShapeDtypeStruct((S, H), jnp.float32),
        scratch_types=[
            pltpu.VMEM((CPW,), jnp.int32),
            pltpu.VMEM((CPW,), jnp.int32),
            pltpu.VMEM((CPW, H), jnp.float32),
            pltpu.VMEM((CPW, H), jnp.float32),
            pltpu.SemaphoreType.DMA,
            pltpu.SemaphoreType.DMA,
        ],
    )
    def _sc_combine(y_hbm, pos_hbm, out_hbm, idx0, idx1, buf0, buf1,
                    sem0, sem1):
        wid = lax.axis_index("s") * NCORES + lax.axis_index("c")
        base = wid * CPW
        pltpu.sync_copy(pos_hbm.at[pl.ds(base, CPW)], idx0)
        pltpu.sync_copy(pos_hbm.at[pl.ds(S + base, CPW)], idx1)
        g0 = pltpu.async_copy(y_hbm.at[idx0], buf0, sem0)
        g1 = pltpu.async_copy(y_hbm.at[idx1], buf1, sem1)
        g0.wait()
        g1.wait()

        def row_body(r, carry):
            for c in range(H // 16):
                sl = pl.ds(c * 16, 16)
                buf0[r, sl] = buf0[r, sl] + buf1[r, sl]
            return carry

        lax.fori_loop(0, CPW, row_body, 0)
        pltpu.sync_copy(buf0, out_hbm.at[pl.ds(base, CPW)])

    return _sc_combine


@jax.jit
def kernel(router_input, hidden_states, router_w, w_gate, w_up, w_down):
    logits, pos, xsc0, xsc1, be, used = pl.pallas_call(
        _route_dispatch_body,
        out_shape=(
            jax.ShapeDtypeStruct((S, E), jnp.float32),
            jax.ShapeDtypeStruct((S, TOPK), jnp.int32),
            jax.ShapeDtypeStruct((S, H), jnp.float32),
            jax.ShapeDtypeStruct((S, H), jnp.float32),
            jax.ShapeDtypeStruct((NB, 1), jnp.int32),
            jax.ShapeDtypeStruct((NB, 1), jnp.int32),
        ),
    )(router_input, router_w, hidden_states)

    pos_flat = jnp.transpose(pos).reshape(P)

    xs = _sc_scatter_kernel()(xsc0, xsc1, pos_flat)

    y = pl.pallas_call(
        _grouped_expert_body,
        grid_spec=pltpu.PrefetchScalarGridSpec(
            num_scalar_prefetch=2,
            grid=(NB,),
            in_specs=[
                pl.BlockSpec((BT, H), lambda b, be_r, u_r: (b, 0)),
                pl.BlockSpec((1, H, FF), lambda b, be_r, u_r: (be_r[b], 0, 0)),
                pl.BlockSpec((1, H, FF), lambda b, be_r, u_r: (be_r[b], 0, 0)),
                pl.BlockSpec((1, FF, H), lambda b, be_r, u_r: (be_r[b], 0, 0)),
            ],
            out_specs=pl.BlockSpec((BT, H), lambda b, be_r, u_r: (b, 0)),
        ),
        out_shape=jax.ShapeDtypeStruct((C, H), jnp.float32),
    )(be.reshape(NB), used.reshape(NB), xs, w_gate, w_up, w_down)

    out = _sc_combine_kernel()(y, pos_flat)
    return (out, logits)


# speedup vs baseline: 3.2880x; 1.0484x over previous
import functools

import jax
import jax.numpy as jnp
from jax import lax
from jax.experimental import pallas as pl
from jax.experimental.pallas import tpu as pltpu
from jax.experimental.pallas import tpu_sc as plsc

S, H, FF, E, TOPK = 2048, 768, 768, 8, 2
P = S * TOPK
BT = 256
NB = P // BT + E
C = NB * BT
RB = 128

NCORES, NSUB = 2, 16
NW = NCORES * NSUB
CPW = S // NW


def _route_dispatch_body(ri_ref, rw_ref, x_ref, logits_ref, pos_ref,
                         xsc0_ref, xsc1_ref, be_ref, used_ref):
    ri = ri_ref[...]
    rw = rw_ref[...]
    logits = lax.dot_general(
        ri, rw, (((1,), (1,)), ((), ())), preferred_element_type=jnp.float32
    )
    logits_ref[...] = logits

    colid = lax.broadcasted_iota(jnp.int32, (S, E), 1)
    m1 = jnp.max(logits, axis=1, keepdims=True)
    idx1 = jnp.min(jnp.where(logits == m1, colid, E), axis=1, keepdims=True)
    l2 = jnp.where(colid == idx1, -jnp.inf, logits)
    m2 = jnp.max(l2, axis=1, keepdims=True)
    idx2 = jnp.min(jnp.where(l2 == m2, colid, E), axis=1, keepdims=True)
    e2 = jnp.exp(m2 - m1)
    denom = 1.0 + e2
    w0 = 1.0 / denom
    w1 = e2 / denom

    x = x_ref[...]

    def pack(v):
        u16 = lax.bitcast_convert_type(v.astype(jnp.bfloat16), jnp.uint16)
        lo = u16[:, :H // 2].astype(jnp.uint32)
        hi = u16[:, H // 2:].astype(jnp.uint32)
        return lax.bitcast_convert_type(lo | (hi << 16), jnp.float32)

    xsc0_ref[...] = pack(x * jnp.sqrt(w0))
    xsc1_ref[...] = pack(x * jnp.sqrt(w1))

    one_hot1 = jnp.where(colid == idx1, 1.0, 0.0)
    one_hot2 = jnp.where(colid == idx2, 1.0, 0.0)
    x12 = jnp.concatenate([one_hot1, one_hot2], axis=1).astype(jnp.bfloat16)

    ri_b = lax.broadcasted_iota(jnp.int32, (RB, RB), 0)
    ci_b = lax.broadcasted_iota(jnp.int32, (RB, RB), 1)
    ls = jnp.where(ci_b < ri_b, 1.0, 0.0).astype(jnp.bfloat16)
    excl_parts = []
    offs = jnp.zeros((1, 2 * E), jnp.float32)
    for blk in range(S // RB):
        sub = x12[blk * RB:(blk + 1) * RB, :]
        part = lax.dot_general(
            ls, sub, (((1,), (0,)), ((), ())),
            preferred_element_type=jnp.float32)
        excl_parts.append(part + offs)
        offs = offs + jnp.sum(sub.astype(jnp.float32), axis=0, keepdims=True)
    excl = jnp.concatenate(excl_parts, axis=0)
    excl1 = excl[:, :E]
    excl2 = excl[:, E:]
    total1 = jnp.sum(one_hot1, axis=0, keepdims=True)
    total2 = jnp.sum(one_hot2, axis=0, keepdims=True)
    counts = total1 + total2

    nblk = jnp.floor((counts + (BT - 1)) * (1.0 / BT))
    er = lax.broadcasted_iota(jnp.int32, (E, E), 0)
    ec = lax.broadcasted_iota(jnp.int32, (E, E), 1)
    m8 = jnp.where(er < ec, 1.0, 0.0)
    blkoff = lax.dot_general(
        nblk, m8, (((1,), (0,)), ((), ())),
        precision=lax.Precision.HIGHEST,
        preferred_element_type=jnp.float32)
    padoff = blkoff * float(BT)
    total_blocks = jnp.sum(nblk, axis=1, keepdims=True)

    def sel(tab, idx):
        t = jnp.broadcast_to(tab, (S, E))
        return jnp.sum(jnp.where(colid == idx, t, 0.0), axis=1, keepdims=True)

    rank0 = jnp.sum(jnp.where(colid == idx1, excl1, 0.0), axis=1,
                    keepdims=True)
    rank1 = jnp.sum(jnp.where(colid == idx2, excl2, 0.0), axis=1,
                    keepdims=True)
    pos0 = sel(padoff, idx1) + rank0
    pos1 = sel(padoff, idx2) + sel(total1, idx2) + rank1
    pos_ref[...] = jnp.concatenate([pos0, pos1], axis=1).astype(jnp.int32)

    br = lax.broadcasted_iota(jnp.int32, (NB, E), 0).astype(jnp.float32)
    bo = jnp.broadcast_to(blkoff, (NB, E))
    be = jnp.sum(jnp.where(bo <= br, 1.0, 0.0), axis=1, keepdims=True) - 1.0
    be_ref[...] = be.astype(jnp.int32)
    brow = lax.broadcasted_iota(jnp.int32, (NB, 1), 0).astype(jnp.float32)
    used_ref[...] = jnp.where(
        brow < jnp.broadcast_to(total_blocks, (NB, 1)), 1, 0
    ).astype(jnp.int32)


def _grouped_expert_body(be_sref, used_sref, x_ref, wg_ref, wu_ref,
                         wd_ref, y_ref):
    b = pl.program_id(0)

    @pl.when(used_sref[b] == 1)
    def _():
        u = lax.bitcast_convert_type(x_ref[...], jnp.uint32)
        lo = lax.bitcast_convert_type(
            u.astype(jnp.uint16), jnp.bfloat16)
        hi = lax.bitcast_convert_type(
            (u >> 16).astype(jnp.uint16), jnp.bfloat16)
        x = jnp.concatenate([lo, hi], axis=1).astype(jnp.float32)
        dims = (((1,), (0,)), ((), ()))
        g = lax.dot_general(x, wg_ref[0], dims,
                            preferred_element_type=jnp.float32)
        u = lax.dot_general(x, wu_ref[0], dims,
                            preferred_element_type=jnp.float32)
        a = jnp.maximum(g, 0.0) * u
        y_ref[...] = lax.dot_general(a, wd_ref[0], dims,
                                     preferred_element_type=jnp.float32)


@functools.lru_cache(maxsize=None)
def _sc_scatter_kernel():
    mesh = plsc.VectorSubcoreMesh(core_axis_name="c", subcore_axis_name="s")

    @functools.partial(
        pl.kernel,
        mesh=mesh,
        out_type=jax.ShapeDtypeStruct((C, H // 2), jnp.float32),
        scratch_types=[
            pltpu.VMEM((CPW,), jnp.int32),
            pltpu.VMEM((CPW,), jnp.int32),
            pltpu.VMEM((CPW, H // 2), jnp.float32),
            pltpu.VMEM((CPW, H // 2), jnp.float32),
            pltpu.SemaphoreType.DMA,
            pltpu.SemaphoreType.DMA,
            pltpu.SemaphoreType.DMA,
        ],
    )
    def _sc_scatter(xsc0_hbm, xsc1_hbm, pos_hbm, out_hbm, idx0, idx1,
                    rows0, rows1, semr0, semr1, semw):
        wid = lax.axis_index("s") * NCORES + lax.axis_index("c")
        base = wid * CPW
        pltpu.sync_copy(pos_hbm.at[pl.ds(base, CPW)], idx0)
        pltpu.sync_copy(pos_hbm.at[pl.ds(S + base, CPW)], idx1)
        r0 = pltpu.async_copy(xsc0_hbm.at[pl.ds(base, CPW)], rows0, semr0)
        r1 = pltpu.async_copy(xsc1_hbm.at[pl.ds(base, CPW)], rows1, semr1)
        r0.wait()
        w0 = pltpu.async_copy(rows0, out_hbm.at[idx0], semw)
        r1.wait()
        w1 = pltpu.async_copy(rows1, out_hbm.at[idx1], semw)
        w0.wait()
        w1.wait()

    return _sc_scatter


@functools.lru_cache(maxsize=None)
def _sc_combine_kernel():
    mesh = plsc.VectorSubcoreMesh(core_axis_name="c", subcore_axis_name="s")

    @functools.partial(
        pl.kernel,
        mesh=mesh,
        out_type=jax.ShapeDtypeStruct((S, H), jnp.float32),
        scratch_types=[
            pltpu.VMEM((CPW,), jnp.int32),
            pltpu.VMEM((CPW,), jnp.int32),
            pltpu.VMEM((CPW, H), jnp.float32),
            pltpu.VMEM((CPW, H), jnp.float32),
            pltpu.SemaphoreType.DMA,
            pltpu.SemaphoreType.DMA,
        ],
    )
    def _sc_combine(y_hbm, pos_hbm, out_hbm, idx0, idx1, buf0, buf1,
                    sem0, sem1):
        wid = lax.axis_index("s") * NCORES + lax.axis_index("c")
        base = wid * CPW
        pltpu.sync_copy(pos_hbm.at[pl.ds(base, CPW)], idx0)
        pltpu.sync_copy(pos_hbm.at[pl.ds(S + base, CPW)], idx1)
        g0 = pltpu.async_copy(y_hbm.at[idx0], buf0, sem0)
        g1 = pltpu.async_copy(y_hbm.at[idx1], buf1, sem1)
        g0.wait()
        g1.wait()

        def row_body(r, carry):
            for c in range(H // 16):
                sl = pl.ds(c * 16, 16)
                buf0[r, sl] = buf0[r, sl] + buf1[r, sl]
            return carry

        lax.fori_loop(0, CPW, row_body, 0)
        pltpu.sync_copy(buf0, out_hbm.at[pl.ds(base, CPW)])

    return _sc_combine


@jax.jit
def kernel(router_input, hidden_states, router_w, w_gate, w_up, w_down):
    logits, pos, xsc0, xsc1, be, used = pl.pallas_call(
        _route_dispatch_body,
        out_shape=(
            jax.ShapeDtypeStruct((S, E), jnp.float32),
            jax.ShapeDtypeStruct((S, TOPK), jnp.int32),
            jax.ShapeDtypeStruct((S, H // 2), jnp.float32),
            jax.ShapeDtypeStruct((S, H // 2), jnp.float32),
            jax.ShapeDtypeStruct((NB, 1), jnp.int32),
            jax.ShapeDtypeStruct((NB, 1), jnp.int32),
        ),
    )(router_input, router_w, hidden_states)

    pos_flat = jnp.transpose(pos).reshape(P)

    xs = _sc_scatter_kernel()(xsc0, xsc1, pos_flat)

    y = pl.pallas_call(
        _grouped_expert_body,
        grid_spec=pltpu.PrefetchScalarGridSpec(
            num_scalar_prefetch=2,
            grid=(NB,),
            in_specs=[
                pl.BlockSpec((BT, H // 2), lambda b, be_r, u_r: (b, 0)),
                pl.BlockSpec((1, H, FF), lambda b, be_r, u_r: (be_r[b], 0, 0)),
                pl.BlockSpec((1, H, FF), lambda b, be_r, u_r: (be_r[b], 0, 0)),
                pl.BlockSpec((1, FF, H), lambda b, be_r, u_r: (be_r[b], 0, 0)),
            ],
            out_specs=pl.BlockSpec((BT, H), lambda b, be_r, u_r: (b, 0)),
        ),
        out_shape=jax.ShapeDtypeStruct((C, H), jnp.float32),
    )(be.reshape(NB), used.reshape(NB), xs, w_gate, w_up, w_down)

    out = _sc_combine_kernel()(y, pos_flat)
    return (out, logits)
